# baseline (speedup 1.0000x reference)
import functools

import jax
import jax.numpy as jnp
from jax import lax
from jax.experimental import pallas as pl
from jax.experimental.pallas import tpu as pltpu
from jax.experimental.pallas import tpu_sc as plsc

_N = 10000
_E = 320000
_D = 128
_DH = 256

_NC = 2
_NS = 16
_NW = _NC * _NS
_C = 128
_CH = -(-_E // (_NW * _C))
_GC = _CH * _NW
_EPAD = _GC * _C
_CH0 = 79
_CH1 = 2 * _CH - _CH0
_NACC = 10112
_ZR = _NACC // _NS
_XR = 632
_XR_LAST = _N - 15 * _XR


def _segsum_body(table, sd, zeros, out, acc, sd_v, rows_v,
                 idx_sem, g_sem, s_sem):
    c = lax.axis_index("c")
    s = lax.axis_index("s")
    start = lax.select(c == 0, s * _CH0, _NS * _CH0 + s * _CH1)
    nck = lax.select(c == 0, jnp.int32(_CH0), jnp.int32(_CH1))

    def idx_start(i, r):
        pltpu.async_copy(sd.at[start + i], sd_v.at[r], idx_sem.at[r])

    def idx_wait(i, r):
        pltpu.make_async_copy(sd.at[start + i], sd_v.at[r],
                              idx_sem.at[r]).wait()

    def gather_start(i, b):
        pltpu.async_copy(table.at[sd_v.at[lax.rem(i, 4), 0]], rows_v.at[b],
                         g_sem.at[b])

    def gather_wait(i, b):
        pltpu.make_async_copy(table.at[sd_v.at[lax.rem(i, 4), 0]],
                              rows_v.at[b], g_sem.at[b]).wait()

    def scat_start(i, b):
        pltpu.async_copy(rows_v.at[b], acc.at[sd_v.at[lax.rem(i, 4), 1]],
                         s_sem.at[b], add=True)

    def scat_wait(i, b):
        pltpu.make_async_copy(rows_v.at[b], acc.at[sd_v.at[lax.rem(i, 4), 1]],
                              s_sem.at[b]).wait()

    idx_start(0, 0)
    pltpu.sync_copy(zeros.at[pl.ds(s * _ZR, _ZR)], acc.at[pl.ds(s * _ZR, _ZR)])
    idx_start(1, 1)
    idx_wait(0, 0)
    gather_start(0, 0)
    plsc.subcore_barrier()

    def chunk(i, carry):
        b = lax.rem(i, 2)
        nb = 1 - b

        @pl.when(i + 1 < nck)
        def _():
            idx_wait(i + 1, lax.rem(i + 1, 4))

            @pl.when(i >= 1)
            def _():
                scat_wait(i - 1, nb)

            gather_start(i + 1, nb)

        @pl.when(i + 2 < nck)
        def _():
            idx_start(i + 2, lax.rem(i + 2, 4))

        gather_wait(i, b)
        scat_start(i, b)
        return carry

    lax.fori_loop(0, nck, chunk, 0)
    scat_wait(nck - 2, lax.rem(nck - 2, 2))
    scat_wait(nck - 1, lax.rem(nck - 1, 2))
    plsc.subcore_barrier()

    @pl.when(s < _NS - 1)
    def _():
        pltpu.sync_copy(acc.at[pl.ds(s * _XR, _XR)], out.at[c, pl.ds(s * _XR, _XR)])

    @pl.when(s == _NS - 1)
    def _():
        pltpu.sync_copy(acc.at[pl.ds(15 * _XR, _XR_LAST)],
                        out.at[c, pl.ds(15 * _XR, _XR_LAST)])


@functools.cache
def _get_segsum():
    return pl.kernel(
        _segsum_body,
        out_type=jax.ShapeDtypeStruct((_NC, _N, _D), jnp.float32),
        mesh=plsc.VectorSubcoreMesh(core_axis_name="c", subcore_axis_name="s",
                                    num_cores=_NC, num_subcores=_NS),
        scratch_types=[
            pltpu.VMEM_SHARED((_NACC, _D), jnp.float32),
            pltpu.VMEM((4, 2, _C), jnp.int32),
            pltpu.VMEM((2, _C, _D), jnp.float32),
            pltpu.SemaphoreType.DMA((4,)),
            pltpu.SemaphoreType.DMA((2,)),
            pltpu.SemaphoreType.DMA((2,)),
        ],
    )


def _segsum(table, sd, zeros):
    return _get_segsum()(table, sd, zeros)


_RB = 1000
_G = _N // _RB


def _enc_body(a0, a1, x, Wr, b, Wo, W2, h_ref, p_ref):
    agg = a0[...] + a1[...]
    h = jnp.dot(agg, Wr[...], preferred_element_type=jnp.float32)
    h = h + jnp.dot(x[...], Wo[...], preferred_element_type=jnp.float32)
    h = jnp.maximum(h + b[...], 0.0)
    h_ref[...] = h
    p_ref[...] = jnp.dot(h, W2[...], preferred_element_type=jnp.float32)


def _tc_fused(a0, a1, x, Wr, b, Wo, W2):
    return pl.pallas_call(
        _enc_body,
        grid=(_G,),
        in_specs=[
            pl.BlockSpec((_RB, _D), lambda i: (i, 0)),
            pl.BlockSpec((_RB, _D), lambda i: (i, 0)),
            pl.BlockSpec((_RB, _D), lambda i: (i, 0)),
            pl.BlockSpec((_D, _DH), lambda i: (0, 0)),
            pl.BlockSpec((1, _DH), lambda i: (0, 0)),
            pl.BlockSpec((_D, _DH), lambda i: (0, 0)),
            pl.BlockSpec((_DH, _D), lambda i: (0, 0)),
        ],
        out_specs=[
            pl.BlockSpec((_RB, _DH), lambda i: (i, 0)),
            pl.BlockSpec((_RB, _D), lambda i: (i, 0)),
        ],
        out_shape=[
            jax.ShapeDtypeStruct((_N, _DH), jnp.float32),
            jax.ShapeDtypeStruct((_N, _D), jnp.float32),
        ],
    )(a0, a1, x, Wr, b.reshape(1, _DH), Wo, W2)


def _post_body(a0, a1, h, Wo, b, o_ref):
    o_ref[...] = (a0[...] + a1[...] + b[...]
                  + jnp.dot(h[...], Wo[...], preferred_element_type=jnp.float32))


def _tc_post(a0, a1, h, Wo, b):
    return pl.pallas_call(
        _post_body,
        grid=(_G,),
        in_specs=[
            pl.BlockSpec((_RB, _D), lambda i: (i, 0)),
            pl.BlockSpec((_RB, _D), lambda i: (i, 0)),
            pl.BlockSpec((_RB, _DH), lambda i: (i, 0)),
            pl.BlockSpec((_DH, _D), lambda i: (0, 0)),
            pl.BlockSpec((1, _D), lambda i: (0, 0)),
        ],
        out_specs=pl.BlockSpec((_RB, _D), lambda i: (i, 0)),
        out_shape=jax.ShapeDtypeStruct((_N, _D), jnp.float32),
    )(a0, a1, h, Wo, b.reshape(1, _D))


def kernel(x, edge_index, We1r, be1, We1o, We2r, be2, We2o,
           Wd1r, bd1, Wd1o, Wd2r, bd2, Wd2o):
    ei = edge_index.astype(jnp.int32)
    pad = _EPAD - _E
    srcs = jnp.concatenate([ei[0], jnp.zeros((pad,), jnp.int32)]).reshape(_GC, _C)
    dsts = jnp.concatenate([ei[1], jnp.full((pad,), _N, jnp.int32)]).reshape(_GC, _C)
    core1 = (jnp.arange(_GC, dtype=jnp.int32) >= _NS * _CH0)[:, None]
    srcs = srcs + core1.astype(jnp.int32) * _N
    sd = jnp.stack([srcs, dsts], axis=1)
    zeros = jnp.zeros((_NACC, _D), jnp.float32)

    def seg(table):
        t2 = jnp.concatenate([table, table], axis=0)
        return _segsum(t2, sd, zeros)

    a = seg(x)
    h, p2 = _tc_fused(a[0], a[1], x, We1r, be1, We1o, We2r)
    a = seg(p2)
    z = _tc_post(a[0], a[1], h, We2o, be2)
    a = seg(z)
    h2, p4 = _tc_fused(a[0], a[1], z, Wd1r, bd1, Wd1o, Wd2r)
    a = seg(p4)
    x_hat = _tc_post(a[0], a[1], h2, Wd2o, bd2)
    return (x_hat, z)


# speedup vs baseline: 1.2433x; 1.2433x over previous
import functools

import jax
import jax.numpy as jnp
from jax import lax
from jax.experimental import pallas as pl
from jax.experimental.pallas import tpu as pltpu
from jax.experimental.pallas import tpu_sc as plsc

_N = 10000
_E = 320000
_D = 128
_DH = 256

_NC = 2
_NS = 16
_NW = _NC * _NS
_C = 128
_CH = -(-_E // (_NW * _C))
_GC = _CH * _NW
_EPAD = _GC * _C
_CH0 = 128
_CH1 = 2 * _CH - _CH0
_NACC = 10112
_ZR = _NACC // _NS
_XR = 632
_XR_LAST = _N - 15 * _XR


def _segsum_body(table, sd, zeros, out, acc, sd_v, rows_v,
                 idx_sem, g_sem, s_sem):
    c = lax.axis_index("c")
    s = lax.axis_index("s")
    start = lax.select(c == 0, s * _CH0, _NS * _CH0 + s * _CH1)
    nck = lax.select(c == 0, jnp.int32(_CH0), jnp.int32(_CH1))

    def idx_start(i, r):
        pltpu.async_copy(sd.at[start + i], sd_v.at[r], idx_sem.at[r])

    def idx_wait(i, r):
        pltpu.make_async_copy(sd.at[start + i], sd_v.at[r],
                              idx_sem.at[r]).wait()

    def gather_start(i, b):
        pltpu.async_copy(table.at[sd_v.at[lax.rem(i, 4), 0]], rows_v.at[b],
                         g_sem.at[b])

    def gather_wait(i, b):
        pltpu.make_async_copy(table.at[sd_v.at[lax.rem(i, 4), 0]],
                              rows_v.at[b], g_sem.at[b]).wait()

    def scat_start(i, b):
        pltpu.async_copy(rows_v.at[b], acc.at[sd_v.at[lax.rem(i, 4), 1]],
                         s_sem.at[b], add=True)

    def scat_wait(i, b):
        pltpu.make_async_copy(rows_v.at[b], acc.at[sd_v.at[lax.rem(i, 4), 1]],
                              s_sem.at[b]).wait()

    idx_start(0, 0)
    pltpu.sync_copy(zeros.at[pl.ds(s * _ZR, _ZR)], acc.at[pl.ds(s * _ZR, _ZR)])
    idx_start(1, 1)
    idx_wait(0, 0)
    gather_start(0, 0)
    plsc.subcore_barrier()

    def chunk(i, carry):
        b = lax.rem(i, 2)
        nb = 1 - b

        @pl.when(i + 1 < nck)
        def _():
            idx_wait(i + 1, lax.rem(i + 1, 4))

            @pl.when(i >= 1)
            def _():
                scat_wait(i - 1, nb)

            gather_start(i + 1, nb)

        @pl.when(i + 2 < nck)
        def _():
            idx_start(i + 2, lax.rem(i + 2, 4))

        gather_wait(i, b)
        scat_start(i, b)
        return carry

    lax.fori_loop(0, nck, chunk, 0)
    scat_wait(nck - 2, lax.rem(nck - 2, 2))
    scat_wait(nck - 1, lax.rem(nck - 1, 2))
    plsc.subcore_barrier()

    @pl.when(s < _NS - 1)
    def _():
        pltpu.sync_copy(acc.at[pl.ds(s * _XR, _XR)], out.at[c, pl.ds(s * _XR, _XR)])

    @pl.when(s == _NS - 1)
    def _():
        pltpu.sync_copy(acc.at[pl.ds(15 * _XR, _XR_LAST)],
                        out.at[c, pl.ds(15 * _XR, _XR_LAST)])


@functools.cache
def _get_segsum():
    return pl.kernel(
        _segsum_body,
        out_type=jax.ShapeDtypeStruct((_NC, _N, _D), jnp.float32),
        mesh=plsc.VectorSubcoreMesh(core_axis_name="c", subcore_axis_name="s",
                                    num_cores=_NC, num_subcores=_NS),
        scratch_types=[
            pltpu.VMEM_SHARED((_NACC, _D), jnp.float32),
            pltpu.VMEM((4, 2, _C), jnp.int32),
            pltpu.VMEM((2, _C, _D), jnp.float32),
            pltpu.SemaphoreType.DMA((4,)),
            pltpu.SemaphoreType.DMA((2,)),
            pltpu.SemaphoreType.DMA((2,)),
        ],
    )


def _segsum(table, sd, zeros):
    return _get_segsum()(table, sd, zeros)


_RB = 1000
_G = _N // _RB


def _enc_body(a0, a1, x, Wr, b, Wo, W2, h_ref, p_ref):
    agg = a0[...] + a1[...]
    h = jnp.dot(agg, Wr[...], preferred_element_type=jnp.float32)
    h = h + jnp.dot(x[...], Wo[...], preferred_element_type=jnp.float32)
    h = jnp.maximum(h + b[...], 0.0)
    h_ref[...] = h
    p_ref[...] = jnp.dot(h, W2[...], preferred_element_type=jnp.float32)


def _tc_fused(a0, a1, x, Wr, b, Wo, W2):
    return pl.pallas_call(
        _enc_body,
        grid=(_G,),
        in_specs=[
            pl.BlockSpec((_RB, _D), lambda i: (i, 0)),
            pl.BlockSpec((_RB, _D), lambda i: (i, 0)),
            pl.BlockSpec((_RB, _D), lambda i: (i, 0)),
            pl.BlockSpec((_D, _DH), lambda i: (0, 0)),
            pl.BlockSpec((1, _DH), lambda i: (0, 0)),
            pl.BlockSpec((_D, _DH), lambda i: (0, 0)),
            pl.BlockSpec((_DH, _D), lambda i: (0, 0)),
        ],
        out_specs=[
            pl.BlockSpec((_RB, _DH), lambda i: (i, 0)),
            pl.BlockSpec((_RB, _D), lambda i: (i, 0)),
        ],
        out_shape=[
            jax.ShapeDtypeStruct((_N, _DH), jnp.float32),
            jax.ShapeDtypeStruct((_N, _D), jnp.float32),
        ],
    )(a0, a1, x, Wr, b.reshape(1, _DH), Wo, W2)


def _post_body(a0, a1, h, Wo, b, o_ref):
    o_ref[...] = (a0[...] + a1[...] + b[...]
                  + jnp.dot(h[...], Wo[...], preferred_element_type=jnp.float32))


def _tc_post(a0, a1, h, Wo, b):
    return pl.pallas_call(
        _post_body,
        grid=(_G,),
        in_specs=[
            pl.BlockSpec((_RB, _D), lambda i: (i, 0)),
            pl.BlockSpec((_RB, _D), lambda i: (i, 0)),
            pl.BlockSpec((_RB, _DH), lambda i: (i, 0)),
            pl.BlockSpec((_DH, _D), lambda i: (0, 0)),
            pl.BlockSpec((1, _D), lambda i: (0, 0)),
        ],
        out_specs=pl.BlockSpec((_RB, _D), lambda i: (i, 0)),
        out_shape=jax.ShapeDtypeStruct((_N, _D), jnp.float32),
    )(a0, a1, h, Wo, b.reshape(1, _D))


def kernel(x, edge_index, We1r, be1, We1o, We2r, be2, We2o,
           Wd1r, bd1, Wd1o, Wd2r, bd2, Wd2o):
    ei = edge_index.astype(jnp.int32)
    pad = _EPAD - _E
    srcs = jnp.concatenate([ei[0], jnp.zeros((pad,), jnp.int32)]).reshape(_GC, _C)
    dsts = jnp.concatenate([ei[1], jnp.full((pad,), _N, jnp.int32)]).reshape(_GC, _C)
    sd = jnp.stack([srcs, dsts], axis=1)
    zeros = jnp.zeros((_NACC, _D), jnp.float32)

    def seg(table):
        return _segsum(table, sd, zeros)

    a = seg(x)
    h, p2 = _tc_fused(a[0], a[1], x, We1r, be1, We1o, We2r)
    a = seg(p2)
    z = _tc_post(a[0], a[1], h, We2o, be2)
    a = seg(z)
    h2, p4 = _tc_fused(a[0], a[1], z, Wd1r, bd1, Wd1o, Wd2r)
    a = seg(p4)
    x_hat = _tc_post(a[0], a[1], h2, Wd2o, bd2)
    return (x_hat, z)


# speedup vs baseline: 1.2841x; 1.0329x over previous
import functools

import jax
import jax.numpy as jnp
from jax import lax
from jax.experimental import pallas as pl
from jax.experimental.pallas import tpu as pltpu
from jax.experimental.pallas import tpu_sc as plsc

_N = 10000
_E = 320000
_D = 128
_DH = 256

_NC = 2
_NS = 16
_NW = _NC * _NS
_C = 128
_CH = -(-_E // (_NW * _C))
_GC = _CH * _NW
_EPAD = _GC * _C
_CH0 = 140
_CH1 = 2 * _CH - _CH0
_NACC = 10112
_ZR = _NACC // _NS
_XR = 632
_XR_LAST = _N - 15 * _XR


def _segsum_body(table, sd, zeros, out, acc, sd_v, rows_v,
                 idx_sem, g_sem, s_sem):
    c = lax.axis_index("c")
    s = lax.axis_index("s")
    start = lax.select(c == 0, s * _CH0, _NS * _CH0 + s * _CH1)
    nck = lax.select(c == 0, jnp.int32(_CH0), jnp.int32(_CH1))

    def idx_start(i, r):
        pltpu.async_copy(sd.at[start + i], sd_v.at[r], idx_sem.at[r])

    def idx_wait(i, r):
        pltpu.make_async_copy(sd.at[start + i], sd_v.at[r],
                              idx_sem.at[r]).wait()

    def gather_start(i, b):
        pltpu.async_copy(table.at[sd_v.at[lax.rem(i, 4), 0]], rows_v.at[b],
                         g_sem.at[b])

    def gather_wait(i, b):
        pltpu.make_async_copy(table.at[sd_v.at[lax.rem(i, 4), 0]],
                              rows_v.at[b], g_sem.at[b]).wait()

    def scat_start(i, b):
        pltpu.async_copy(rows_v.at[b], acc.at[sd_v.at[lax.rem(i, 4), 1]],
                         s_sem.at[b], add=True)

    def scat_wait(i, b):
        pltpu.make_async_copy(rows_v.at[b], acc.at[sd_v.at[lax.rem(i, 4), 1]],
                              s_sem.at[b]).wait()

    idx_start(0, 0)
    pltpu.sync_copy(zeros.at[pl.ds(s * _ZR, _ZR)], acc.at[pl.ds(s * _ZR, _ZR)])
    idx_start(1, 1)
    idx_wait(0, 0)
    gather_start(0, 0)
    plsc.subcore_barrier()

    def chunk(i, carry):
        b = lax.rem(i, 2)
        nb = 1 - b

        @pl.when(i + 1 < nck)
        def _():
            idx_wait(i + 1, lax.rem(i + 1, 4))

            @pl.when(i >= 1)
            def _():
                scat_wait(i - 1, nb)

            gather_start(i + 1, nb)

        @pl.when(i + 2 < nck)
        def _():
            idx_start(i + 2, lax.rem(i + 2, 4))

        gather_wait(i, b)
        scat_start(i, b)
        return carry

    lax.fori_loop(0, nck, chunk, 0)
    scat_wait(nck - 2, lax.rem(nck - 2, 2))
    scat_wait(nck - 1, lax.rem(nck - 1, 2))
    plsc.subcore_barrier()

    @pl.when(s < _NS - 1)
    def _():
        pltpu.sync_copy(acc.at[pl.ds(s * _XR, _XR)], out.at[c, pl.ds(s * _XR, _XR)])

    @pl.when(s == _NS - 1)
    def _():
        pltpu.sync_copy(acc.at[pl.ds(15 * _XR, _XR_LAST)],
                        out.at[c, pl.ds(15 * _XR, _XR_LAST)])


@functools.cache
def _get_segsum():
    return pl.kernel(
        _segsum_body,
        out_type=jax.ShapeDtypeStruct((_NC, _N, _D), jnp.float32),
        mesh=plsc.VectorSubcoreMesh(core_axis_name="c", subcore_axis_name="s",
                                    num_cores=_NC, num_subcores=_NS),
        scratch_types=[
            pltpu.VMEM_SHARED((_NACC, _D), jnp.float32),
            pltpu.VMEM((4, 2, _C), jnp.int32),
            pltpu.VMEM((2, _C, _D), jnp.float32),
            pltpu.SemaphoreType.DMA((4,)),
            pltpu.SemaphoreType.DMA((2,)),
            pltpu.SemaphoreType.DMA((2,)),
        ],
    )


def _segsum(table, sd, zeros):
    return _get_segsum()(table, sd, zeros)


_RB = 1000
_G = _N // _RB


def _enc_body(a0, a1, x, Wr, b, Wo, W2, h_ref, p_ref):
    agg = a0[...] + a1[...]
    h = jnp.dot(agg, Wr[...], preferred_element_type=jnp.float32)
    h = h + jnp.dot(x[...], Wo[...], preferred_element_type=jnp.float32)
    h = jnp.maximum(h + b[...], 0.0)
    h_ref[...] = h
    p_ref[...] = jnp.dot(h, W2[...], preferred_element_type=jnp.float32)


def _tc_fused(a0, a1, x, Wr, b, Wo, W2):
    return pl.pallas_call(
        _enc_body,
        grid=(_G,),
        in_specs=[
            pl.BlockSpec((_RB, _D), lambda i: (i, 0)),
            pl.BlockSpec((_RB, _D), lambda i: (i, 0)),
            pl.BlockSpec((_RB, _D), lambda i: (i, 0)),
            pl.BlockSpec((_D, _DH), lambda i: (0, 0)),
            pl.BlockSpec((1, _DH), lambda i: (0, 0)),
            pl.BlockSpec((_D, _DH), lambda i: (0, 0)),
            pl.BlockSpec((_DH, _D), lambda i: (0, 0)),
        ],
        out_specs=[
            pl.BlockSpec((_RB, _DH), lambda i: (i, 0)),
            pl.BlockSpec((_RB, _D), lambda i: (i, 0)),
        ],
        out_shape=[
            jax.ShapeDtypeStruct((_N, _DH), jnp.float32),
            jax.ShapeDtypeStruct((_N, _D), jnp.float32),
        ],
    )(a0, a1, x, Wr, b.reshape(1, _DH), Wo, W2)


def _post_body(a0, a1, h, Wo, b, o_ref):
    o_ref[...] = (a0[...] + a1[...] + b[...]
                  + jnp.dot(h[...], Wo[...], preferred_element_type=jnp.float32))


def _tc_post(a0, a1, h, Wo, b):
    return pl.pallas_call(
        _post_body,
        grid=(_G,),
        in_specs=[
            pl.BlockSpec((_RB, _D), lambda i: (i, 0)),
            pl.BlockSpec((_RB, _D), lambda i: (i, 0)),
            pl.BlockSpec((_RB, _DH), lambda i: (i, 0)),
            pl.BlockSpec((_DH, _D), lambda i: (0, 0)),
            pl.BlockSpec((1, _D), lambda i: (0, 0)),
        ],
        out_specs=pl.BlockSpec((_RB, _D), lambda i: (i, 0)),
        out_shape=jax.ShapeDtypeStruct((_N, _D), jnp.float32),
    )(a0, a1, h, Wo, b.reshape(1, _D))


def kernel(x, edge_index, We1r, be1, We1o, We2r, be2, We2o,
           Wd1r, bd1, Wd1o, Wd2r, bd2, Wd2o):
    ei = edge_index.astype(jnp.int32)
    pad = _EPAD - _E
    srcs = jnp.concatenate([ei[0], jnp.zeros((pad,), jnp.int32)]).reshape(_GC, _C)
    dsts = jnp.concatenate([ei[1], jnp.full((pad,), _N, jnp.int32)]).reshape(_GC, _C)
    sd = jnp.stack([srcs, dsts], axis=1)
    zeros = jnp.zeros((_NACC, _D), jnp.float32)

    def seg(table):
        return _segsum(table, sd, zeros)

    a = seg(x)
    h, p2 = _tc_fused(a[0], a[1], x, We1r, be1, We1o, We2r)
    a = seg(p2)
    z = _tc_post(a[0], a[1], h, We2o, be2)
    a = seg(z)
    h2, p4 = _tc_fused(a[0], a[1], z, Wd1r, bd1, Wd1o, Wd2r)
    a = seg(p4)
    x_hat = _tc_post(a[0], a[1], h2, Wd2o, bd2)
    return (x_hat, z)
